# baseline (speedup 1.0000x reference)
import jax
import jax.numpy as jnp
from jax.experimental import pallas as pl
from jax.experimental.pallas import tpu as pltpu


def _mm_kernel(x_ref, y1_ref, y2_ref, o_ref):
    TN = y1_ref.shape[2]
    xb = x_ref[0]
    o_ref[0, :, :TN] = jnp.dot(xb, y1_ref[0], preferred_element_type=jnp.float32)
    o_ref[0, :, TN:] = jnp.dot(xb, y2_ref[0], preferred_element_type=jnp.float32)


def kernel(Y, x):
    B, N, _ = Y.shape
    _, C, F, _ = x.shape
    M = C * F
    x2 = x.reshape(B, M, N).astype(jnp.bfloat16)
    Yb = Y.astype(jnp.bfloat16)
    TN = 1024
    out = pl.pallas_call(
        _mm_kernel,
        grid=(B,),
        in_specs=[
            pl.BlockSpec((1, M, N), lambda b: (b, 0, 0)),
            pl.BlockSpec((1, N, TN), lambda b: (b, 0, 0)),
            pl.BlockSpec((1, N, TN), lambda b: (b, 0, 1)),
        ],
        out_specs=pl.BlockSpec((1, M, N), lambda b: (b, 0, 0)),
        out_shape=jax.ShapeDtypeStruct((B, M, N), jnp.float32),
        compiler_params=pltpu.CompilerParams(
            allow_input_fusion=[False, True, True],
        ),
    )(x2, Yb, Yb)
    return out.reshape(B, C, F, N)


# speedup vs baseline: 2.0364x; 2.0364x over previous
import jax
import jax.numpy as jnp
from jax.experimental import pallas as pl
from jax.experimental.pallas import tpu as pltpu


def _mm_kernel(x_ref, y1_ref, y2_ref, o_ref):
    TN = y1_ref.shape[2]
    xb = x_ref[0]
    o_ref[0, :, :TN] = jnp.dot(xb, y1_ref[0], preferred_element_type=jnp.float32)
    o_ref[0, :, TN:] = jnp.dot(xb, y2_ref[0], preferred_element_type=jnp.float32)


def kernel(Y, x):
    B, N, _ = Y.shape
    _, C, F, _ = x.shape
    M = C * F
    x2 = x.reshape(B, M, N).astype(jnp.bfloat16)
    Yb = Y.astype(jnp.bfloat16)
    TN = 512
    out = pl.pallas_call(
        _mm_kernel,
        grid=(B, N // (2 * TN)),
        in_specs=[
            pl.BlockSpec((1, M, N), lambda b, j: (b, 0, 0)),
            pl.BlockSpec((1, N, TN), lambda b, j: (b, 0, 2 * j)),
            pl.BlockSpec((1, N, TN), lambda b, j: (b, 0, 2 * j + 1)),
        ],
        out_specs=pl.BlockSpec((1, M, 2 * TN), lambda b, j: (b, 0, j)),
        out_shape=jax.ShapeDtypeStruct((B, M, N), jnp.float32),
        compiler_params=pltpu.CompilerParams(
            allow_input_fusion=[False, True, True],
        ),
    )(x2, Yb, Yb)
    return out.reshape(B, C, F, N)
